# baseline (speedup 1.0000x reference)
import functools

import jax
import jax.numpy as jnp
from jax import lax
from jax.experimental import pallas as pl
from jax.experimental.pallas import tpu as pltpu
from jax.experimental.pallas import tpu_sc as plsc

B = 16384
NUM_V = 1000000
F = 26
D = 16
NC = 2
NS = 16
NW = NC * NS
BPW = B // NW
C = 64
NCHUNK = BPW // C
IPC = C * F
NSTREAM = IPC // 128


UNT_W = 2048
UNT_NCH = 488
UNT_REM_ROWS = NUM_V - UNT_NCH * UNT_W
UNT_ORPC = UNT_W // 8
UNT_PW = -(-UNT_NCH // NW)


def _untile_sc_body(tabT_hbm, tail_hbm, flat_hbm, buf_i, buf_o):
    wid = lax.axis_index("s") * NC + lax.axis_index("c")
    rowi = lax.iota(jnp.int32, 16)

    @pl.when(wid == 0)
    def _():
        pltpu.sync_copy(tail_hbm, flat_hbm.at[pl.ds(UNT_NCH * UNT_ORPC,
                                                    UNT_REM_ROWS * D // 128)])

    def chunk(k, carry):
        cidx = k * NW + wid

        @pl.when(cidx < UNT_NCH)
        def _():
            colbase = pl.multiple_of(cidx * UNT_W, UNT_W)
            for d in range(D):
                pltpu.sync_copy(tabT_hbm.at[d, pl.ds(colbase, UNT_W)],
                                buf_i.at[d, pl.ds(0, UNT_W)])

            ones = jnp.full((16,), 1, jnp.int32)

            def body(c8, colv):
                for dc in range(8):
                    e = plsc.load_gather(buf_i, [rowi, colv])
                    buf_o[c8, pl.ds(dc * D, D)] = e
                    colv = colv + ones
                return colv

            lax.fori_loop(0, UNT_ORPC, body, jnp.zeros((16,), jnp.int32),
                          unroll=4)
            pltpu.sync_copy(buf_o,
                            flat_hbm.at[pl.ds(
                                pl.multiple_of(cidx * UNT_ORPC, UNT_ORPC),
                                UNT_ORPC)])
        return carry

    lax.fori_loop(0, UNT_PW, chunk, 0)


_untile_call = pl.kernel(
    _untile_sc_body,
    out_type=jax.ShapeDtypeStruct((NUM_V * D // 128, 128), jnp.float32),
    mesh=plsc.VectorSubcoreMesh(core_axis_name="c", subcore_axis_name="s",
                                num_cores=NC, num_subcores=NS),
    compiler_params=pltpu.CompilerParams(use_tc_tiling_on_sc=True,
                                         needs_layout_passes=False),
    scratch_types=[
        pltpu.VMEM((D, UNT_W + 24), jnp.float32),
        pltpu.VMEM((UNT_ORPC, 128), jnp.float32),
    ],
)


def _fm_sc_body(feat_hbm, val_hbm, table_hbm, fm_hbm, idx_v, val_v, rows_v,
                fm_v, sem):
    wid = lax.axis_index("s") * NC + lax.axis_index("c")
    for c in range(NCHUNK):
        pltpu.sync_copy(feat_hbm.at[wid, c], idx_v)
        pltpu.sync_copy(val_hbm.at[wid, c], val_v)
        copies = [
            pltpu.async_copy(table_hbm.at[idx_v.at[j]],
                             rows_v.at[pl.ds(j * 128, 128)], sem)
            for j in range(NSTREAM)
        ]
        for cp in copies:
            cp.wait()

        def body(b, carry):
            base = b * F
            vv0 = val_v[b, 0:16]
            vv1 = val_v[b, 16:32]
            s = jnp.zeros((D,), jnp.float32)
            q = jnp.zeros((D,), jnp.float32)
            for f in range(F):
                v = vv0[f] if f < 16 else vv1[f - 16]
                e = rows_v[base + f, :]
                t = e * v
                s = s + t
                q = q + t * t
            fm_v[b, :] = 0.5 * (s * s - q)
            return carry

        lax.fori_loop(0, C, body, 0)
        pltpu.sync_copy(fm_v, fm_hbm.at[pl.ds(wid * BPW + c * C, C)])


_fm_call = pl.kernel(
    _fm_sc_body,
    out_type=jax.ShapeDtypeStruct((B, D), jnp.float32),
    mesh=plsc.VectorSubcoreMesh(core_axis_name="c", subcore_axis_name="s",
                                num_cores=NC, num_subcores=NS),
    compiler_params=pltpu.CompilerParams(use_tc_tiling_on_sc=False),
    scratch_types=[
        pltpu.VMEM((NSTREAM, 128), jnp.int32),
        pltpu.VMEM((C, 32), jnp.float32),
        pltpu.VMEM((IPC, D), jnp.float32),
        pltpu.VMEM((C, D), jnp.float32),
        pltpu.SemaphoreType.DMA,
    ],
)

BLK = 2048


def _mlp_tc_body(fm_ref, w1_ref, b1_ref, w2_ref, b2_ref, wp_ref, bp_ref,
                 out_ref):
    h = jnp.maximum(jnp.dot(fm_ref[...], w1_ref[...],
                            preferred_element_type=jnp.float32)
                    + b1_ref[...], 0.0)
    h = jnp.maximum(jnp.dot(h, w2_ref[...],
                            preferred_element_type=jnp.float32)
                    + b2_ref[...], 0.0)
    o = jnp.sum(h * wp_ref[...].reshape(1, -1), axis=1) + bp_ref[0, 0]
    out_ref[0, 0, :] = o


_mlp_call = pl.pallas_call(
    _mlp_tc_body,
    grid=(B // BLK,),
    in_specs=[
        pl.BlockSpec((BLK, D), lambda i: (i, 0)),
        pl.BlockSpec((D, 64), lambda i: (0, 0)),
        pl.BlockSpec((1, 64), lambda i: (0, 0)),
        pl.BlockSpec((64, 32), lambda i: (0, 0)),
        pl.BlockSpec((1, 32), lambda i: (0, 0)),
        pl.BlockSpec((32, 1), lambda i: (0, 0)),
        pl.BlockSpec((1, 1), lambda i: (0, 0)),
    ],
    out_specs=pl.BlockSpec((1, 1, BLK), lambda i: (i, 0, 0)),
    out_shape=jax.ShapeDtypeStruct((B // BLK, 1, BLK), jnp.float32),
)


def kernel(features, feature_values, emb_table, W1, b1, W2, b2, Wp, bp):
    feat_r = features.reshape(NW, NCHUNK, NSTREAM, 128)
    val_pad = jnp.pad(feature_values, ((0, 0), (0, 32 - F)))
    val_r = val_pad.reshape(NW, NCHUNK, C, 32)
    tail = emb_table[UNT_NCH * UNT_W:].reshape(UNT_REM_ROWS * D // 128, 128)
    table_dense = _untile_call(emb_table.T, tail).reshape(NUM_V, D)
    fm = _fm_call(feat_r, val_r, table_dense)
    out = _mlp_call(fm, W1, b1.reshape(1, -1), W2, b2.reshape(1, -1), Wp,
                    bp.reshape(1, 1))
    return out.reshape(-1)


# speedup vs baseline: 1.2411x; 1.2411x over previous
import functools

import jax
import jax.numpy as jnp
from jax import lax
from jax.experimental import pallas as pl
from jax.experimental.pallas import tpu as pltpu
from jax.experimental.pallas import tpu_sc as plsc

B = 16384
NUM_V = 1000000
F = 26
D = 16
NC = 2
NS = 16
NW = NC * NS
BPW = B // NW
C = 64
NCHUNK = BPW // C
IPC = C * F
NSTREAM = IPC // 128


UNT_W = 2048
UNT_NCH = 488
UNT_REM_ROWS = NUM_V - UNT_NCH * UNT_W
UNT_ORPC = UNT_W // 8
UNT_PW = -(-UNT_NCH // NW)


def _untile_sc_body(tabT_hbm, tail_hbm, flat_hbm, buf_i, buf_o, dsem):
    wid = lax.axis_index("s") * NC + lax.axis_index("c")
    rowi = lax.iota(jnp.int32, 16)

    @pl.when(wid == 0)
    def _():
        pltpu.sync_copy(tail_hbm, flat_hbm.at[pl.ds(UNT_NCH * UNT_ORPC,
                                                    UNT_REM_ROWS * D // 128)])

    def chunk(k, carry):
        cidx = k * NW + wid

        @pl.when(cidx < UNT_NCH)
        def _():
            colbase = pl.multiple_of(cidx * UNT_W, UNT_W)
            cps = [pltpu.async_copy(tabT_hbm.at[d, pl.ds(colbase, UNT_W)],
                                    buf_i.at[d, pl.ds(0, UNT_W)], dsem)
                   for d in range(D)]
            for cp in cps:
                cp.wait()

            ones = jnp.full((16,), 1, jnp.int32)

            def body(c8, colv):
                for dc in range(8):
                    e = plsc.load_gather(buf_i, [rowi, colv])
                    buf_o[c8, pl.ds(dc * D, D)] = e
                    colv = colv + ones
                return colv

            lax.fori_loop(0, UNT_ORPC, body, jnp.zeros((16,), jnp.int32),
                          unroll=4)
            pltpu.sync_copy(buf_o,
                            flat_hbm.at[pl.ds(
                                pl.multiple_of(cidx * UNT_ORPC, UNT_ORPC),
                                UNT_ORPC)])
        return carry

    lax.fori_loop(0, UNT_PW, chunk, 0)


_untile_call = pl.kernel(
    _untile_sc_body,
    out_type=jax.ShapeDtypeStruct((NUM_V * D // 128, 128), jnp.float32),
    mesh=plsc.VectorSubcoreMesh(core_axis_name="c", subcore_axis_name="s",
                                num_cores=NC, num_subcores=NS),
    compiler_params=pltpu.CompilerParams(use_tc_tiling_on_sc=True,
                                         needs_layout_passes=False),
    scratch_types=[
        pltpu.VMEM((D, UNT_W + 24), jnp.float32),
        pltpu.VMEM((UNT_ORPC, 128), jnp.float32),
        pltpu.SemaphoreType.DMA,
    ],
)


def _fm_sc_body(feat_hbm, val_hbm, table_hbm, fm_hbm, idx_v, val_v, rows_v,
                fm_v, sem):
    wid = lax.axis_index("s") * NC + lax.axis_index("c")
    for c in range(NCHUNK):
        pltpu.sync_copy(feat_hbm.at[wid, c], idx_v)
        pltpu.sync_copy(val_hbm.at[wid, c], val_v)
        copies = [
            pltpu.async_copy(table_hbm.at[idx_v.at[j]],
                             rows_v.at[pl.ds(j * 128, 128)], sem)
            for j in range(NSTREAM)
        ]
        for cp in copies:
            cp.wait()

        def body(b, carry):
            base = b * F
            vv0 = val_v[b, 0:16]
            vv1 = val_v[b, 16:32]
            s = jnp.zeros((D,), jnp.float32)
            q = jnp.zeros((D,), jnp.float32)
            for f in range(F):
                v = vv0[f] if f < 16 else vv1[f - 16]
                e = rows_v[base + f, :]
                t = e * v
                s = s + t
                q = q + t * t
            fm_v[b, :] = 0.5 * (s * s - q)
            return carry

        lax.fori_loop(0, C, body, 0)
        pltpu.sync_copy(fm_v, fm_hbm.at[pl.ds(wid * BPW + c * C, C)])


_fm_call = pl.kernel(
    _fm_sc_body,
    out_type=jax.ShapeDtypeStruct((B, D), jnp.float32),
    mesh=plsc.VectorSubcoreMesh(core_axis_name="c", subcore_axis_name="s",
                                num_cores=NC, num_subcores=NS),
    compiler_params=pltpu.CompilerParams(use_tc_tiling_on_sc=False),
    scratch_types=[
        pltpu.VMEM((NSTREAM, 128), jnp.int32),
        pltpu.VMEM((C, 32), jnp.float32),
        pltpu.VMEM((IPC, D), jnp.float32),
        pltpu.VMEM((C, D), jnp.float32),
        pltpu.SemaphoreType.DMA,
    ],
)

BLK = 2048


def _mlp_tc_body(fm_ref, w1_ref, b1_ref, w2_ref, b2_ref, wp_ref, bp_ref,
                 out_ref):
    h = jnp.maximum(jnp.dot(fm_ref[...], w1_ref[...],
                            preferred_element_type=jnp.float32)
                    + b1_ref[...], 0.0)
    h = jnp.maximum(jnp.dot(h, w2_ref[...],
                            preferred_element_type=jnp.float32)
                    + b2_ref[...], 0.0)
    o = jnp.sum(h * wp_ref[...].reshape(1, -1), axis=1) + bp_ref[0, 0]
    out_ref[0, 0, :] = o


_mlp_call = pl.pallas_call(
    _mlp_tc_body,
    grid=(B // BLK,),
    in_specs=[
        pl.BlockSpec((BLK, D), lambda i: (i, 0)),
        pl.BlockSpec((D, 64), lambda i: (0, 0)),
        pl.BlockSpec((1, 64), lambda i: (0, 0)),
        pl.BlockSpec((64, 32), lambda i: (0, 0)),
        pl.BlockSpec((1, 32), lambda i: (0, 0)),
        pl.BlockSpec((32, 1), lambda i: (0, 0)),
        pl.BlockSpec((1, 1), lambda i: (0, 0)),
    ],
    out_specs=pl.BlockSpec((1, 1, BLK), lambda i: (i, 0, 0)),
    out_shape=jax.ShapeDtypeStruct((B // BLK, 1, BLK), jnp.float32),
)


def kernel(features, feature_values, emb_table, W1, b1, W2, b2, Wp, bp):
    feat_r = features.reshape(NW, NCHUNK, NSTREAM, 128)
    val_pad = jnp.pad(feature_values, ((0, 0), (0, 32 - F)))
    val_r = val_pad.reshape(NW, NCHUNK, C, 32)
    tail = emb_table[UNT_NCH * UNT_W:].reshape(UNT_REM_ROWS * D // 128, 128)
    table_dense = _untile_call(emb_table.T, tail).reshape(NUM_V, D)
    fm = _fm_call(feat_r, val_r, table_dense)
    out = _mlp_call(fm, W1, b1.reshape(1, -1), W2, b2.reshape(1, -1), Wp,
                    bp.reshape(1, 1))
    return out.reshape(-1)


# speedup vs baseline: 1.8621x; 1.5004x over previous
import functools

import jax
import jax.numpy as jnp
from jax import lax
from jax.experimental import pallas as pl
from jax.experimental.pallas import tpu as pltpu
from jax.experimental.pallas import tpu_sc as plsc

B = 16384
NUM_V = 1000000
F = 26
D = 16
NC = 2
NS = 16
NW = NC * NS
BPW = B // NW
C = 64
NCHUNK = BPW // C
IPC = C * F
NSTREAM = IPC // 128


UNT_W = 2048
UNT_NCH = 488
UNT_REM_ROWS = NUM_V - UNT_NCH * UNT_W
UNT_ORPC = UNT_W // 8
UNT_PW = -(-UNT_NCH // NW)


def _untile_sc_body(tabT_hbm, tail_hbm, flat_hbm, buf_i, buf_o, dsem):
    wid = lax.axis_index("s") * NC + lax.axis_index("c")
    rowi = lax.iota(jnp.int32, 16)

    @pl.when(wid == 0)
    def _():
        pltpu.sync_copy(tail_hbm, flat_hbm.at[pl.ds(UNT_NCH * UNT_ORPC,
                                                    UNT_REM_ROWS * D // 128)])

    def chunk(k, carry):
        cidx = k * NW + wid

        @pl.when(cidx < UNT_NCH)
        def _():
            colbase = pl.multiple_of(cidx * UNT_W, UNT_W)
            cps = [pltpu.async_copy(tabT_hbm.at[d, pl.ds(colbase, UNT_W)],
                                    buf_i.at[d, pl.ds(0, UNT_W)], dsem)
                   for d in range(D)]
            for cp in cps:
                cp.wait()

            ones = jnp.full((16,), 1, jnp.int32)

            @plsc.parallel_loop(0, UNT_ORPC, unroll=4,
                                carry=jnp.zeros((16,), jnp.int32))
            def body(c8, colv):
                for dc in range(8):
                    e = plsc.load_gather(buf_i, [rowi, colv])
                    buf_o[c8, pl.ds(dc * D, D)] = e
                    colv = colv + ones
                return colv
            pltpu.sync_copy(buf_o,
                            flat_hbm.at[pl.ds(
                                pl.multiple_of(cidx * UNT_ORPC, UNT_ORPC),
                                UNT_ORPC)])
        return carry

    lax.fori_loop(0, UNT_PW, chunk, 0)


_untile_call = pl.kernel(
    _untile_sc_body,
    out_type=jax.ShapeDtypeStruct((NUM_V * D // 128, 128), jnp.float32),
    mesh=plsc.VectorSubcoreMesh(core_axis_name="c", subcore_axis_name="s",
                                num_cores=NC, num_subcores=NS),
    compiler_params=pltpu.CompilerParams(use_tc_tiling_on_sc=True,
                                         needs_layout_passes=False),
    scratch_types=[
        pltpu.VMEM((D, UNT_W + 24), jnp.float32),
        pltpu.VMEM((UNT_ORPC, 128), jnp.float32),
        pltpu.SemaphoreType.DMA,
    ],
)


def _fm_sc_body(feat_hbm, val_hbm, table_hbm, fm_hbm, idx_v, val_v, rows_v,
                fm_v, sem):
    wid = lax.axis_index("s") * NC + lax.axis_index("c")
    for c in range(NCHUNK):
        pltpu.sync_copy(feat_hbm.at[wid, c], idx_v)
        pltpu.sync_copy(val_hbm.at[wid, c], val_v)
        copies = [
            pltpu.async_copy(table_hbm.at[idx_v.at[j]],
                             rows_v.at[pl.ds(j * 128, 128)], sem)
            for j in range(NSTREAM)
        ]
        for cp in copies:
            cp.wait()

        def body(b, carry):
            base = b * F
            vv0 = val_v[b, 0:16]
            vv1 = val_v[b, 16:32]
            s = jnp.zeros((D,), jnp.float32)
            q = jnp.zeros((D,), jnp.float32)
            for f in range(F):
                v = vv0[f] if f < 16 else vv1[f - 16]
                e = rows_v[base + f, :]
                t = e * v
                s = s + t
                q = q + t * t
            fm_v[b, :] = 0.5 * (s * s - q)
            return carry

        lax.fori_loop(0, C, body, 0)
        pltpu.sync_copy(fm_v, fm_hbm.at[pl.ds(wid * BPW + c * C, C)])


_fm_call = pl.kernel(
    _fm_sc_body,
    out_type=jax.ShapeDtypeStruct((B, D), jnp.float32),
    mesh=plsc.VectorSubcoreMesh(core_axis_name="c", subcore_axis_name="s",
                                num_cores=NC, num_subcores=NS),
    compiler_params=pltpu.CompilerParams(use_tc_tiling_on_sc=False),
    scratch_types=[
        pltpu.VMEM((NSTREAM, 128), jnp.int32),
        pltpu.VMEM((C, 32), jnp.float32),
        pltpu.VMEM((IPC, D), jnp.float32),
        pltpu.VMEM((C, D), jnp.float32),
        pltpu.SemaphoreType.DMA,
    ],
)

BLK = 2048


def _mlp_tc_body(fm_ref, w1_ref, b1_ref, w2_ref, b2_ref, wp_ref, bp_ref,
                 out_ref):
    h = jnp.maximum(jnp.dot(fm_ref[...], w1_ref[...],
                            preferred_element_type=jnp.float32)
                    + b1_ref[...], 0.0)
    h = jnp.maximum(jnp.dot(h, w2_ref[...],
                            preferred_element_type=jnp.float32)
                    + b2_ref[...], 0.0)
    o = jnp.sum(h * wp_ref[...].reshape(1, -1), axis=1) + bp_ref[0, 0]
    out_ref[0, 0, :] = o


_mlp_call = pl.pallas_call(
    _mlp_tc_body,
    grid=(B // BLK,),
    in_specs=[
        pl.BlockSpec((BLK, D), lambda i: (i, 0)),
        pl.BlockSpec((D, 64), lambda i: (0, 0)),
        pl.BlockSpec((1, 64), lambda i: (0, 0)),
        pl.BlockSpec((64, 32), lambda i: (0, 0)),
        pl.BlockSpec((1, 32), lambda i: (0, 0)),
        pl.BlockSpec((32, 1), lambda i: (0, 0)),
        pl.BlockSpec((1, 1), lambda i: (0, 0)),
    ],
    out_specs=pl.BlockSpec((1, 1, BLK), lambda i: (i, 0, 0)),
    out_shape=jax.ShapeDtypeStruct((B // BLK, 1, BLK), jnp.float32),
)


def kernel(features, feature_values, emb_table, W1, b1, W2, b2, Wp, bp):
    feat_r = features.reshape(NW, NCHUNK, NSTREAM, 128)
    val_pad = jnp.pad(feature_values, ((0, 0), (0, 32 - F)))
    val_r = val_pad.reshape(NW, NCHUNK, C, 32)
    tail = emb_table[UNT_NCH * UNT_W:].reshape(UNT_REM_ROWS * D // 128, 128)
    table_dense = _untile_call(emb_table.T, tail).reshape(NUM_V, D)
    fm = _fm_call(feat_r, val_r, table_dense)
    out = _mlp_call(fm, W1, b1.reshape(1, -1), W2, b2.reshape(1, -1), Wp,
                    bp.reshape(1, 1))
    return out.reshape(-1)


# speedup vs baseline: 1.8655x; 1.0018x over previous
import functools

import jax
import jax.numpy as jnp
from jax import lax
from jax.experimental import pallas as pl
from jax.experimental.pallas import tpu as pltpu
from jax.experimental.pallas import tpu_sc as plsc

B = 16384
NUM_V = 1000000
F = 26
D = 16
NC = 2
NS = 16
NW = NC * NS
BPW = B // NW
C = 64
NCHUNK = BPW // C
IPC = C * F
NSTREAM = IPC // 128


UNT_W = 2048
UNT_NCH = 488
UNT_REM_ROWS = NUM_V - UNT_NCH * UNT_W
UNT_ORPC = UNT_W // 8
UNT_PW = -(-UNT_NCH // NW)


def _untile_sc_body(tabT_hbm, tail_hbm, flat_hbm, buf_i, buf_o, dsem):
    wid = lax.axis_index("s") * NC + lax.axis_index("c")
    rowi = lax.iota(jnp.int32, 16)

    @pl.when(wid == 0)
    def _():
        pltpu.sync_copy(tail_hbm, flat_hbm.at[pl.ds(UNT_NCH * UNT_ORPC,
                                                    UNT_REM_ROWS * D // 128)])

    def chunk(k, carry):
        cidx = k * NW + wid

        @pl.when(cidx < UNT_NCH)
        def _():
            colbase = pl.multiple_of(cidx * UNT_W, UNT_W)
            cps = [pltpu.async_copy(tabT_hbm.at[d, pl.ds(colbase, UNT_W)],
                                    buf_i.at[d, pl.ds(0, UNT_W)], dsem)
                   for d in range(D)]
            for cp in cps:
                cp.wait()

            ones = jnp.full((16,), 1, jnp.int32)

            @plsc.parallel_loop(0, UNT_ORPC, unroll=4,
                                carry=jnp.zeros((16,), jnp.int32))
            def body(c8, colv):
                for dc in range(8):
                    e = plsc.load_gather(buf_i, [rowi, colv])
                    buf_o[c8, pl.ds(dc * D, D)] = e
                    colv = colv + ones
                return colv
            pltpu.sync_copy(buf_o,
                            flat_hbm.at[pl.ds(
                                pl.multiple_of(cidx * UNT_ORPC, UNT_ORPC),
                                UNT_ORPC)])
        return carry

    lax.fori_loop(0, UNT_PW, chunk, 0)


_untile_call = pl.kernel(
    _untile_sc_body,
    out_type=jax.ShapeDtypeStruct((NUM_V * D // 128, 128), jnp.float32),
    mesh=plsc.VectorSubcoreMesh(core_axis_name="c", subcore_axis_name="s",
                                num_cores=NC, num_subcores=NS),
    compiler_params=pltpu.CompilerParams(use_tc_tiling_on_sc=True,
                                         needs_layout_passes=False),
    scratch_types=[
        pltpu.VMEM((D, UNT_W + 24), jnp.float32),
        pltpu.VMEM((UNT_ORPC, 128), jnp.float32),
        pltpu.SemaphoreType.DMA,
    ],
)


def _fm_sc_body(feat_hbm, val_hbm, table_hbm, fm_hbm, idx_v, val_v, rows_v,
                fm_v, sem):
    wid = lax.axis_index("s") * NC + lax.axis_index("c")
    for c in range(NCHUNK):
        pltpu.sync_copy(feat_hbm.at[wid, c], idx_v)
        pltpu.sync_copy(val_hbm.at[wid, c], val_v)
        copies = [
            pltpu.async_copy(table_hbm.at[idx_v.at[j]],
                             rows_v.at[pl.ds(j * 128, 128)], sem)
            for j in range(NSTREAM)
        ]
        for cp in copies:
            cp.wait()

        @plsc.parallel_loop(0, C, unroll=2)
        def body(b):
            base = b * F
            vv0 = val_v[b, 0:16]
            vv1 = val_v[b, 16:32]
            s = jnp.zeros((D,), jnp.float32)
            q = jnp.zeros((D,), jnp.float32)
            for f in range(F):
                v = vv0[f] if f < 16 else vv1[f - 16]
                e = rows_v[base + f, :]
                t = e * v
                s = s + t
                q = q + t * t
            fm_v[b, :] = 0.5 * (s * s - q)
        pltpu.sync_copy(fm_v, fm_hbm.at[pl.ds(wid * BPW + c * C, C)])


_fm_call = pl.kernel(
    _fm_sc_body,
    out_type=jax.ShapeDtypeStruct((B, D), jnp.float32),
    mesh=plsc.VectorSubcoreMesh(core_axis_name="c", subcore_axis_name="s",
                                num_cores=NC, num_subcores=NS),
    compiler_params=pltpu.CompilerParams(use_tc_tiling_on_sc=False),
    scratch_types=[
        pltpu.VMEM((NSTREAM, 128), jnp.int32),
        pltpu.VMEM((C, 32), jnp.float32),
        pltpu.VMEM((IPC, D), jnp.float32),
        pltpu.VMEM((C, D), jnp.float32),
        pltpu.SemaphoreType.DMA,
    ],
)

BLK = 2048


def _mlp_tc_body(fm_ref, w1_ref, b1_ref, w2_ref, b2_ref, wp_ref, bp_ref,
                 out_ref):
    h = jnp.maximum(jnp.dot(fm_ref[...], w1_ref[...],
                            preferred_element_type=jnp.float32)
                    + b1_ref[...], 0.0)
    h = jnp.maximum(jnp.dot(h, w2_ref[...],
                            preferred_element_type=jnp.float32)
                    + b2_ref[...], 0.0)
    o = jnp.sum(h * wp_ref[...].reshape(1, -1), axis=1) + bp_ref[0, 0]
    out_ref[0, 0, :] = o


_mlp_call = pl.pallas_call(
    _mlp_tc_body,
    grid=(B // BLK,),
    in_specs=[
        pl.BlockSpec((BLK, D), lambda i: (i, 0)),
        pl.BlockSpec((D, 64), lambda i: (0, 0)),
        pl.BlockSpec((1, 64), lambda i: (0, 0)),
        pl.BlockSpec((64, 32), lambda i: (0, 0)),
        pl.BlockSpec((1, 32), lambda i: (0, 0)),
        pl.BlockSpec((32, 1), lambda i: (0, 0)),
        pl.BlockSpec((1, 1), lambda i: (0, 0)),
    ],
    out_specs=pl.BlockSpec((1, 1, BLK), lambda i: (i, 0, 0)),
    out_shape=jax.ShapeDtypeStruct((B // BLK, 1, BLK), jnp.float32),
)


def kernel(features, feature_values, emb_table, W1, b1, W2, b2, Wp, bp):
    feat_r = features.reshape(NW, NCHUNK, NSTREAM, 128)
    val_pad = jnp.pad(feature_values, ((0, 0), (0, 32 - F)))
    val_r = val_pad.reshape(NW, NCHUNK, C, 32)
    tail = emb_table[UNT_NCH * UNT_W:].reshape(UNT_REM_ROWS * D // 128, 128)
    table_dense = _untile_call(emb_table.T, tail).reshape(NUM_V, D)
    fm = _fm_call(feat_r, val_r, table_dense)
    out = _mlp_call(fm, W1, b1.reshape(1, -1), W2, b2.reshape(1, -1), Wp,
                    bp.reshape(1, 1))
    return out.reshape(-1)


# speedup vs baseline: 1.8681x; 1.0014x over previous
import functools

import jax
import jax.numpy as jnp
from jax import lax
from jax.experimental import pallas as pl
from jax.experimental.pallas import tpu as pltpu
from jax.experimental.pallas import tpu_sc as plsc

B = 16384
NUM_V = 1000000
F = 26
D = 16
NC = 2
NS = 16
NW = NC * NS
BPW = B // NW
C = 64
NCHUNK = BPW // C
IPC = C * F
NSTREAM = IPC // 128


UNT_W = 2048
UNT_NCH = 488
UNT_REM_ROWS = NUM_V - UNT_NCH * UNT_W
UNT_ORPC = UNT_W // 8
UNT_PW = -(-UNT_NCH // NW)


def _untile_sc_body(tabT_hbm, tail_hbm, flat_hbm, buf_i, buf_o, dsem):
    wid = lax.axis_index("s") * NC + lax.axis_index("c")
    rowi = lax.iota(jnp.int32, 16)

    @pl.when(wid == 0)
    def _():
        pltpu.sync_copy(tail_hbm, flat_hbm.at[pl.ds(UNT_NCH * UNT_ORPC,
                                                    UNT_REM_ROWS * D // 128)])

    def chunk(k, carry):
        cidx = k * NW + wid

        @pl.when(cidx < UNT_NCH)
        def _():
            colbase = pl.multiple_of(cidx * UNT_W, UNT_W)
            pltpu.sync_copy(tabT_hbm.at[:, pl.ds(colbase, UNT_W)], buf_i)

            ones = jnp.full((16,), 1, jnp.int32)

            @plsc.parallel_loop(0, UNT_ORPC, unroll=8,
                                carry=jnp.zeros((16,), jnp.int32))
            def body(c8, colv):
                for dc in range(8):
                    e = plsc.load_gather(buf_i, [rowi, colv])
                    buf_o[c8, pl.ds(dc * D, D)] = e
                    colv = colv + ones
                return colv
            pltpu.sync_copy(buf_o,
                            flat_hbm.at[pl.ds(
                                pl.multiple_of(cidx * UNT_ORPC, UNT_ORPC),
                                UNT_ORPC)])
        return carry

    lax.fori_loop(0, UNT_PW, chunk, 0)


_untile_call = pl.kernel(
    _untile_sc_body,
    out_type=jax.ShapeDtypeStruct((NUM_V * D // 128, 128), jnp.float32),
    mesh=plsc.VectorSubcoreMesh(core_axis_name="c", subcore_axis_name="s",
                                num_cores=NC, num_subcores=NS),
    compiler_params=pltpu.CompilerParams(use_tc_tiling_on_sc=True,
                                         needs_layout_passes=False),
    scratch_types=[
        pltpu.VMEM((D, UNT_W), jnp.float32),
        pltpu.VMEM((UNT_ORPC, 128), jnp.float32),
        pltpu.SemaphoreType.DMA,
    ],
)


def _fm_sc_body(feat_hbm, val_hbm, table_hbm, fm_hbm, idx_v, val_v, rows_v,
                fm_v, sem):
    wid = lax.axis_index("s") * NC + lax.axis_index("c")
    for c in range(NCHUNK):
        pltpu.sync_copy(feat_hbm.at[wid, c], idx_v)
        pltpu.sync_copy(val_hbm.at[wid, c], val_v)
        copies = [
            pltpu.async_copy(table_hbm.at[idx_v.at[j]],
                             rows_v.at[pl.ds(j * 128, 128)], sem)
            for j in range(NSTREAM)
        ]
        for cp in copies:
            cp.wait()

        @plsc.parallel_loop(0, C, unroll=2)
        def body(b):
            base = b * F
            vv0 = val_v[b, 0:16]
            vv1 = val_v[b, 16:32]
            s = jnp.zeros((D,), jnp.float32)
            q = jnp.zeros((D,), jnp.float32)
            for f in range(F):
                v = vv0[f] if f < 16 else vv1[f - 16]
                e = rows_v[base + f, :]
                t = e * v
                s = s + t
                q = q + t * t
            fm_v[b, :] = 0.5 * (s * s - q)
        pltpu.sync_copy(fm_v, fm_hbm.at[pl.ds(wid * BPW + c * C, C)])


_fm_call = pl.kernel(
    _fm_sc_body,
    out_type=jax.ShapeDtypeStruct((B, D), jnp.float32),
    mesh=plsc.VectorSubcoreMesh(core_axis_name="c", subcore_axis_name="s",
                                num_cores=NC, num_subcores=NS),
    compiler_params=pltpu.CompilerParams(use_tc_tiling_on_sc=False),
    scratch_types=[
        pltpu.VMEM((NSTREAM, 128), jnp.int32),
        pltpu.VMEM((C, 32), jnp.float32),
        pltpu.VMEM((IPC, D), jnp.float32),
        pltpu.VMEM((C, D), jnp.float32),
        pltpu.SemaphoreType.DMA,
    ],
)

BLK = 2048


def _mlp_tc_body(fm_ref, w1_ref, b1_ref, w2_ref, b2_ref, wp_ref, bp_ref,
                 out_ref):
    h = jnp.maximum(jnp.dot(fm_ref[...], w1_ref[...],
                            preferred_element_type=jnp.float32)
                    + b1_ref[...], 0.0)
    h = jnp.maximum(jnp.dot(h, w2_ref[...],
                            preferred_element_type=jnp.float32)
                    + b2_ref[...], 0.0)
    o = jnp.sum(h * wp_ref[...].reshape(1, -1), axis=1) + bp_ref[0, 0]
    out_ref[0, 0, :] = o


_mlp_call = pl.pallas_call(
    _mlp_tc_body,
    grid=(B // BLK,),
    in_specs=[
        pl.BlockSpec((BLK, D), lambda i: (i, 0)),
        pl.BlockSpec((D, 64), lambda i: (0, 0)),
        pl.BlockSpec((1, 64), lambda i: (0, 0)),
        pl.BlockSpec((64, 32), lambda i: (0, 0)),
        pl.BlockSpec((1, 32), lambda i: (0, 0)),
        pl.BlockSpec((32, 1), lambda i: (0, 0)),
        pl.BlockSpec((1, 1), lambda i: (0, 0)),
    ],
    out_specs=pl.BlockSpec((1, 1, BLK), lambda i: (i, 0, 0)),
    out_shape=jax.ShapeDtypeStruct((B // BLK, 1, BLK), jnp.float32),
)


def kernel(features, feature_values, emb_table, W1, b1, W2, b2, Wp, bp):
    feat_r = features.reshape(NW, NCHUNK, NSTREAM, 128)
    val_pad = jnp.pad(feature_values, ((0, 0), (0, 32 - F)))
    val_r = val_pad.reshape(NW, NCHUNK, C, 32)
    tail = emb_table[UNT_NCH * UNT_W:].reshape(UNT_REM_ROWS * D // 128, 128)
    table_dense = _untile_call(emb_table.T, tail).reshape(NUM_V, D)
    fm = _fm_call(feat_r, val_r, table_dense)
    out = _mlp_call(fm, W1, b1.reshape(1, -1), W2, b2.reshape(1, -1), Wp,
                    bp.reshape(1, 1))
    return out.reshape(-1)


# speedup vs baseline: 1.9364x; 1.0365x over previous
import functools

import jax
import jax.numpy as jnp
from jax import lax
from jax.experimental import pallas as pl
from jax.experimental.pallas import tpu as pltpu
from jax.experimental.pallas import tpu_sc as plsc

B = 16384
NUM_V = 1000000
F = 26
D = 16
NC = 2
NS = 16
NW = NC * NS
BPW = B // NW
C = 64
NCHUNK = BPW // C
IPC = C * F
NSTREAM = IPC // 128


UNT_W = 2048
UNT_NCH = 488
UNT_REM_ROWS = NUM_V - UNT_NCH * UNT_W
UNT_ORPC = UNT_W // 8
UNT_PW = -(-UNT_NCH // NW)


def _untile_sc_body(tabT_hbm, tail_hbm, flat_hbm, buf_i, buf_o, dsem):
    wid = lax.axis_index("s") * NC + lax.axis_index("c")
    rowi = lax.iota(jnp.int32, 16)

    @pl.when(wid == 0)
    def _():
        pltpu.sync_copy(tail_hbm, flat_hbm.at[pl.ds(UNT_NCH * UNT_ORPC,
                                                    UNT_REM_ROWS * D // 128)])

    def chunk(k, carry):
        cidx = k * NW + wid

        @pl.when(cidx < UNT_NCH)
        def _():
            colbase = pl.multiple_of(cidx * UNT_W, UNT_W)
            pltpu.sync_copy(tabT_hbm.at[:, pl.ds(colbase, UNT_W)], buf_i)

            @pl.when(k > 0)
            def _():
                pltpu.make_async_copy(flat_hbm.at[pl.ds(0, UNT_ORPC)],
                                      buf_o, dsem).wait()

            ones = jnp.full((16,), 1, jnp.int32)

            @plsc.parallel_loop(0, UNT_ORPC, unroll=8,
                                carry=jnp.zeros((16,), jnp.int32))
            def body(c8, colv):
                for dc in range(8):
                    e = plsc.load_gather(buf_i, [rowi, colv])
                    buf_o[c8, pl.ds(dc * D, D)] = e
                    colv = colv + ones
                return colv
            pltpu.async_copy(buf_o,
                             flat_hbm.at[pl.ds(
                                 pl.multiple_of(cidx * UNT_ORPC, UNT_ORPC),
                                 UNT_ORPC)], dsem)
        return carry

    lax.fori_loop(0, UNT_PW, chunk, 0)
    pltpu.make_async_copy(flat_hbm.at[pl.ds(0, UNT_ORPC)], buf_o, dsem).wait()


_untile_call = pl.kernel(
    _untile_sc_body,
    out_type=jax.ShapeDtypeStruct((NUM_V * D // 128, 128), jnp.float32),
    mesh=plsc.VectorSubcoreMesh(core_axis_name="c", subcore_axis_name="s",
                                num_cores=NC, num_subcores=NS),
    compiler_params=pltpu.CompilerParams(use_tc_tiling_on_sc=True,
                                         needs_layout_passes=False),
    scratch_types=[
        pltpu.VMEM((D, UNT_W), jnp.float32),
        pltpu.VMEM((UNT_ORPC, 128), jnp.float32),
        pltpu.SemaphoreType.DMA,
    ],
)


def _fm_sc_body(feat_hbm, val_hbm, table_hbm, fm_hbm, idx_v, val_v, rows_v,
                fm_v, sem):
    wid = lax.axis_index("s") * NC + lax.axis_index("c")
    for c in range(NCHUNK):
        pltpu.sync_copy(feat_hbm.at[wid, c], idx_v)
        pltpu.sync_copy(val_hbm.at[wid, c], val_v)
        copies = [
            pltpu.async_copy(table_hbm.at[idx_v.at[j]],
                             rows_v.at[pl.ds(j * 128, 128)], sem)
            for j in range(NSTREAM)
        ]
        for cp in copies:
            cp.wait()

        @plsc.parallel_loop(0, C, unroll=2)
        def body(b):
            base = b * F
            vv0 = val_v[b, 0:16]
            vv1 = val_v[b, 16:32]
            s = jnp.zeros((D,), jnp.float32)
            q = jnp.zeros((D,), jnp.float32)
            for f in range(F):
                v = vv0[f] if f < 16 else vv1[f - 16]
                e = rows_v[base + f, :]
                t = e * v
                s = s + t
                q = q + t * t
            fm_v[b, :] = 0.5 * (s * s - q)
        pltpu.sync_copy(fm_v, fm_hbm.at[pl.ds(wid * BPW + c * C, C)])


_fm_call = pl.kernel(
    _fm_sc_body,
    out_type=jax.ShapeDtypeStruct((B, D), jnp.float32),
    mesh=plsc.VectorSubcoreMesh(core_axis_name="c", subcore_axis_name="s",
                                num_cores=NC, num_subcores=NS),
    compiler_params=pltpu.CompilerParams(use_tc_tiling_on_sc=False),
    scratch_types=[
        pltpu.VMEM((NSTREAM, 128), jnp.int32),
        pltpu.VMEM((C, 32), jnp.float32),
        pltpu.VMEM((IPC, D), jnp.float32),
        pltpu.VMEM((C, D), jnp.float32),
        pltpu.SemaphoreType.DMA,
    ],
)

BLK = 2048


def _mlp_tc_body(fm_ref, w1_ref, b1_ref, w2_ref, b2_ref, wp_ref, bp_ref,
                 out_ref):
    h = jnp.maximum(jnp.dot(fm_ref[...], w1_ref[...],
                            preferred_element_type=jnp.float32)
                    + b1_ref[...], 0.0)
    h = jnp.maximum(jnp.dot(h, w2_ref[...],
                            preferred_element_type=jnp.float32)
                    + b2_ref[...], 0.0)
    o = jnp.sum(h * wp_ref[...].reshape(1, -1), axis=1) + bp_ref[0, 0]
    out_ref[0, 0, :] = o


_mlp_call = pl.pallas_call(
    _mlp_tc_body,
    grid=(B // BLK,),
    in_specs=[
        pl.BlockSpec((BLK, D), lambda i: (i, 0)),
        pl.BlockSpec((D, 64), lambda i: (0, 0)),
        pl.BlockSpec((1, 64), lambda i: (0, 0)),
        pl.BlockSpec((64, 32), lambda i: (0, 0)),
        pl.BlockSpec((1, 32), lambda i: (0, 0)),
        pl.BlockSpec((32, 1), lambda i: (0, 0)),
        pl.BlockSpec((1, 1), lambda i: (0, 0)),
    ],
    out_specs=pl.BlockSpec((1, 1, BLK), lambda i: (i, 0, 0)),
    out_shape=jax.ShapeDtypeStruct((B // BLK, 1, BLK), jnp.float32),
)


def kernel(features, feature_values, emb_table, W1, b1, W2, b2, Wp, bp):
    feat_r = features.reshape(NW, NCHUNK, NSTREAM, 128)
    val_pad = jnp.pad(feature_values, ((0, 0), (0, 32 - F)))
    val_r = val_pad.reshape(NW, NCHUNK, C, 32)
    tail = emb_table[UNT_NCH * UNT_W:].reshape(UNT_REM_ROWS * D // 128, 128)
    table_dense = _untile_call(emb_table.T, tail).reshape(NUM_V, D)
    fm = _fm_call(feat_r, val_r, table_dense)
    out = _mlp_call(fm, W1, b1.reshape(1, -1), W2, b2.reshape(1, -1), Wp,
                    bp.reshape(1, 1))
    return out.reshape(-1)
